# Initial kernel scaffold; baseline (speedup 1.0000x reference)
import functools

import jax
import jax.numpy as jnp
from jax import lax
from jax.experimental import pallas as pl
from jax.experimental.pallas import tpu as pltpu
from jax.experimental.pallas import tpu_sc as plsc

NC = 2
NS = 16
NW = NC * NS
CH = 128


def _proj_body(x_ref, w_ref, o_ref):
    o_ref[...] = jnp.dot(x_ref[...], w_ref[...],
                         preferred_element_type=jnp.float32)


def _mid_body(a_ref, d_ref, w1_ref, q_ref, degc_ref):
    s = a_ref[0] + a_ref[1]
    deg = d_ref[0] + d_ref[1]
    degc = jnp.maximum(deg, 1.0)
    z = jnp.tanh(jnp.tanh(s / degc))
    h1 = jnp.where(z >= 0, z, 0.2 * z)
    q_ref[...] = jnp.dot(h1, w1_ref[...], preferred_element_type=jnp.float32)
    degc_ref[...] = degc


def _final_body(b_ref, degc_ref, batch_ref, wc_ref, bc_ref, o_ref):
    s = b_ref[0] + b_ref[1]
    z = jnp.tanh(jnp.tanh(s / degc_ref[...]))
    h2 = jnp.where(z >= 0, z, 0.2 * z)
    n = batch_ref.shape[1]
    g = o_ref.shape[0]
    gids = lax.broadcasted_iota(jnp.int32, (g, n), 0)
    onehot_t = (gids == batch_ref[...]).astype(jnp.float32)
    pooled = jnp.dot(onehot_t, h2, preferred_element_type=jnp.float32)
    cnt = jnp.sum(onehot_t, axis=1, keepdims=True)
    pooled = pooled / jnp.maximum(cnt, 1.0)
    o_ref[...] = (jnp.dot(pooled, wc_ref[...],
                          preferred_element_type=jnp.float32)
                  + bc_ref[...])


def _edge_pass(n, npad, nch, with_deg):
    zpt = npad // NS
    opt = n // NS

    mesh = plsc.VectorSubcoreMesh(core_axis_name="c", subcore_axis_name="s")
    out_type = [jax.ShapeDtypeStruct((NC, n, 16), jnp.float32)]
    scratch = [
        pltpu.VMEM_SHARED((npad, 16), jnp.float32),
        pltpu.VMEM((nch, CH), jnp.int32),
        pltpu.VMEM((nch, CH), jnp.int32),
        pltpu.VMEM((CH, 16), jnp.float32),
        pltpu.VMEM((zpt, 16), jnp.float32),
        pltpu.SemaphoreType.DMA,
    ]
    if with_deg:
        out_type.append(jax.ShapeDtypeStruct((NC, n, 16), jnp.float32))
        scratch.insert(1, pltpu.VMEM_SHARED((npad, 16), jnp.float32))
        scratch.insert(-1, pltpu.VMEM((CH, 16), jnp.float32))

    def body_deg(table_h, src_h, dst_h, z_h, ones_h, acc_h, deg_h,
                 acc_s, deg_s, srcb, dstb, rows, zbuf, onesb, sem):
        cid = lax.axis_index("c")
        sid = lax.axis_index("s")
        pltpu.sync_copy(src_h.at[cid, sid], srcb)
        pltpu.sync_copy(dst_h.at[cid, sid], dstb)
        pltpu.sync_copy(z_h, zbuf)
        pltpu.sync_copy(ones_h, onesb)
        r0 = sid * zpt
        pltpu.sync_copy(zbuf, acc_s.at[pl.ds(r0, zpt)])
        pltpu.sync_copy(zbuf, deg_s.at[pl.ds(r0, zpt)])
        plsc.subcore_barrier()

        def step(j, carry):
            pltpu.async_copy(table_h.at[srcb.at[j]], rows, sem).wait()
            pltpu.sync_copy(rows, acc_s.at[dstb.at[j]], add=True)
            pltpu.sync_copy(onesb, deg_s.at[dstb.at[j]], add=True)
            return carry

        lax.fori_loop(0, nch, step, 0)
        plsc.subcore_barrier()
        o0 = sid * opt
        pltpu.sync_copy(acc_s.at[pl.ds(o0, opt)], acc_h.at[cid, pl.ds(o0, opt)])
        pltpu.sync_copy(deg_s.at[pl.ds(o0, opt)], deg_h.at[cid, pl.ds(o0, opt)])

    def body_nodeg(table_h, src_h, dst_h, z_h, acc_h,
                   acc_s, srcb, dstb, rows, zbuf, sem):
        cid = lax.axis_index("c")
        sid = lax.axis_index("s")
        pltpu.sync_copy(src_h.at[cid, sid], srcb)
        pltpu.sync_copy(dst_h.at[cid, sid], dstb)
        pltpu.sync_copy(z_h, zbuf)
        r0 = sid * zpt
        pltpu.sync_copy(zbuf, acc_s.at[pl.ds(r0, zpt)])
        plsc.subcore_barrier()

        def step(j, carry):
            pltpu.async_copy(table_h.at[srcb.at[j]], rows, sem).wait()
            pltpu.sync_copy(rows, acc_s.at[dstb.at[j]], add=True)
            return carry

        lax.fori_loop(0, nch, step, 0)
        plsc.subcore_barrier()
        o0 = sid * opt
        pltpu.sync_copy(acc_s.at[pl.ds(o0, opt)], acc_h.at[cid, pl.ds(o0, opt)])

    body = body_deg if with_deg else body_nodeg
    return pl.kernel(body, out_type=tuple(out_type), mesh=mesh,
                     scratch_types=tuple(scratch))


@functools.lru_cache(maxsize=None)
def _edge_pass_cached(n, npad, nch, with_deg):
    return _edge_pass(n, npad, nch, with_deg)


def kernel(x, edge_index, batch, W0, W1, Wc, bc):
    n, d = x.shape
    e = edge_index.shape[1]
    nq = W0.shape[1]
    g = 64
    out_dim = Wc.shape[1]

    per = NW * CH
    e_pad = ((e + per - 1) // per) * per
    nch = e_pad // per
    npad = ((n + NS - 1) // NS) * NS + NS

    src = edge_index[0]
    dst = edge_index[1]
    pad = e_pad - e
    if pad:
        src = jnp.concatenate([src, jnp.zeros((pad,), jnp.int32)])
        dst = jnp.concatenate([dst, jnp.full((pad,), n, jnp.int32)])
    src = src.reshape(NC, NS, nch, CH)
    dst = dst.reshape(NC, NS, nch, CH)
    zeros_st = jnp.zeros((npad // NS, nq), jnp.float32)
    ones_st = jnp.ones((CH, nq), jnp.float32)

    p = pl.pallas_call(
        _proj_body,
        out_shape=jax.ShapeDtypeStruct((n, nq), jnp.float32),
    )(x, W0)

    acc1, deg1 = _edge_pass_cached(n, npad, nch, True)(
        p, src, dst, zeros_st, ones_st)

    q, degc = pl.pallas_call(
        _mid_body,
        out_shape=(jax.ShapeDtypeStruct((n, nq), jnp.float32),
                   jax.ShapeDtypeStruct((n, nq), jnp.float32)),
    )(acc1, deg1, W1)

    (acc2,) = _edge_pass_cached(n, npad, nch, False)(q, src, dst, zeros_st)

    out = pl.pallas_call(
        _final_body,
        out_shape=jax.ShapeDtypeStruct((g, out_dim), jnp.float32),
    )(acc2, degc, batch.reshape(1, n), Wc, bc.reshape(1, out_dim))
    return out


# speedup vs baseline: 11.4547x; 11.4547x over previous
import functools

import jax
import jax.numpy as jnp
from jax import lax
from jax.experimental import pallas as pl
from jax.experimental.pallas import tpu as pltpu
from jax.experimental.pallas import tpu_sc as plsc

NC = 2
NS = 16
NW = NC * NS
CH = 128


def _proj_body(x_ref, w_ref, o_ref):
    o_ref[...] = jnp.dot(x_ref[...], w_ref[...],
                         preferred_element_type=jnp.float32)


def _mid_body(a_ref, d_ref, w1_ref, q_ref, degc_ref):
    s = a_ref[0] + a_ref[1]
    deg = d_ref[0] + d_ref[1]
    degc = jnp.maximum(deg, 1.0)
    z = jnp.tanh(jnp.tanh(s / degc))
    h1 = jnp.where(z >= 0, z, 0.2 * z)
    q_ref[...] = jnp.dot(h1, w1_ref[...], preferred_element_type=jnp.float32)
    degc_ref[...] = degc


def _final_body(b_ref, degc_ref, batch_ref, wc_ref, bc_ref, o_ref):
    s = b_ref[0] + b_ref[1]
    z = jnp.tanh(jnp.tanh(s / degc_ref[...]))
    h2 = jnp.where(z >= 0, z, 0.2 * z)
    n = batch_ref.shape[1]
    g = o_ref.shape[0]
    gids = lax.broadcasted_iota(jnp.int32, (g, n), 0)
    onehot_t = (gids == batch_ref[...]).astype(jnp.float32)
    pooled = jnp.dot(onehot_t, h2, preferred_element_type=jnp.float32)
    cnt = jnp.sum(onehot_t, axis=1, keepdims=True)
    pooled = pooled / jnp.maximum(cnt, 1.0)
    o_ref[...] = (jnp.dot(pooled, wc_ref[...],
                          preferred_element_type=jnp.float32)
                  + bc_ref[...])


def _edge_pass(n, npad, nch, with_deg):
    zpt = npad // NS

    mesh = plsc.VectorSubcoreMesh(core_axis_name="c", subcore_axis_name="s")
    out_type = [jax.ShapeDtypeStruct((NC, npad, 16), jnp.float32)]
    scratch = [
        pltpu.VMEM_SHARED((npad, 16), jnp.float32),
        pltpu.VMEM((nch, CH), jnp.int32),
        pltpu.VMEM((nch, CH), jnp.int32),
        pltpu.VMEM((CH, 16), jnp.float32),
        pltpu.VMEM((zpt, 16), jnp.float32),
        pltpu.SemaphoreType.DMA,
    ]
    if with_deg:
        out_type.append(jax.ShapeDtypeStruct((NC, npad, 16), jnp.float32))
        scratch.insert(1, pltpu.VMEM_SHARED((npad, 16), jnp.float32))
        scratch.insert(-1, pltpu.VMEM((CH, 16), jnp.float32))

    def body_deg(table_h, src_h, dst_h, z_h, ones_h, acc_h, deg_h,
                 acc_s, deg_s, srcb, dstb, rows, zbuf, onesb, sem):
        cid = lax.axis_index("c")
        sid = lax.axis_index("s")
        pltpu.sync_copy(src_h.at[cid, sid], srcb)
        pltpu.sync_copy(dst_h.at[cid, sid], dstb)
        pltpu.sync_copy(z_h, zbuf)
        pltpu.sync_copy(ones_h, onesb)
        r0 = sid * zpt
        pltpu.sync_copy(zbuf, acc_s.at[pl.ds(r0, zpt)])
        pltpu.sync_copy(zbuf, deg_s.at[pl.ds(r0, zpt)])
        plsc.subcore_barrier()

        def step(j, carry):
            pltpu.async_copy(table_h.at[srcb.at[j]], rows, sem).wait()
            pltpu.sync_copy(rows, acc_s.at[dstb.at[j]], add=True)
            pltpu.sync_copy(onesb, deg_s.at[dstb.at[j]], add=True)
            return carry

        lax.fori_loop(0, nch, step, 0)
        plsc.subcore_barrier()
        pltpu.sync_copy(acc_s.at[pl.ds(r0, zpt)], acc_h.at[cid, pl.ds(r0, zpt)])
        pltpu.sync_copy(deg_s.at[pl.ds(r0, zpt)], deg_h.at[cid, pl.ds(r0, zpt)])

    def body_nodeg(table_h, src_h, dst_h, z_h, acc_h,
                   acc_s, srcb, dstb, rows, zbuf, sem):
        cid = lax.axis_index("c")
        sid = lax.axis_index("s")
        pltpu.sync_copy(src_h.at[cid, sid], srcb)
        pltpu.sync_copy(dst_h.at[cid, sid], dstb)
        pltpu.sync_copy(z_h, zbuf)
        r0 = sid * zpt
        pltpu.sync_copy(zbuf, acc_s.at[pl.ds(r0, zpt)])
        plsc.subcore_barrier()

        def step(j, carry):
            pltpu.async_copy(table_h.at[srcb.at[j]], rows, sem).wait()
            pltpu.sync_copy(rows, acc_s.at[dstb.at[j]], add=True)
            return carry

        lax.fori_loop(0, nch, step, 0)
        plsc.subcore_barrier()
        pltpu.sync_copy(acc_s.at[pl.ds(r0, zpt)], acc_h.at[cid, pl.ds(r0, zpt)])

    body = body_deg if with_deg else body_nodeg
    return pl.kernel(body, out_type=tuple(out_type), mesh=mesh,
                     scratch_types=tuple(scratch),
                     compiler_params=pltpu.CompilerParams(
                         use_tc_tiling_on_sc=False))


@functools.lru_cache(maxsize=None)
def _edge_pass_cached(n, npad, nch, with_deg):
    return _edge_pass(n, npad, nch, with_deg)


def kernel(x, edge_index, batch, W0, W1, Wc, bc):
    n, d = x.shape
    e = edge_index.shape[1]
    nq = W0.shape[1]
    g = 64
    out_dim = Wc.shape[1]

    per = NW * CH
    e_pad = ((e + per - 1) // per) * per
    nch = e_pad // per
    rpt = 8 * NS
    npad = ((n + rpt - 1) // rpt) * rpt
    if npad == n:
        npad += rpt

    src = edge_index[0]
    dst = edge_index[1]
    pad = e_pad - e
    if pad:
        src = jnp.concatenate([src, jnp.zeros((pad,), jnp.int32)])
        dst = jnp.concatenate([dst, jnp.full((pad,), n, jnp.int32)])
    src = src.reshape(NC, NS, nch, CH)
    dst = dst.reshape(NC, NS, nch, CH)
    zeros_st = jnp.zeros((npad // NS, nq), jnp.float32)
    ones_st = jnp.ones((CH, nq), jnp.float32)

    p = pl.pallas_call(
        _proj_body,
        out_shape=jax.ShapeDtypeStruct((n, nq), jnp.float32),
    )(x, W0)

    acc1, deg1 = _edge_pass_cached(n, npad, nch, True)(
        p, src, dst, zeros_st, ones_st)
    acc1 = acc1[:, :n]
    deg1 = deg1[:, :n]

    q, degc = pl.pallas_call(
        _mid_body,
        out_shape=(jax.ShapeDtypeStruct((n, nq), jnp.float32),
                   jax.ShapeDtypeStruct((n, nq), jnp.float32)),
    )(acc1, deg1, W1)

    (acc2,) = _edge_pass_cached(n, npad, nch, False)(q, src, dst, zeros_st)
    acc2 = acc2[:, :n]

    out = pl.pallas_call(
        _final_body,
        out_shape=jax.ShapeDtypeStruct((g, out_dim), jnp.float32),
    )(acc2, degc, batch.reshape(1, n), Wc, bc.reshape(1, out_dim))
    return out
